# baseline (speedup 1.0000x reference)
import dataclasses
import functools

import jax
import jax.numpy as jnp
from jax import lax
from jax.experimental import pallas as pl
from jax.experimental.pallas import tpu as pltpu
from jax.experimental.pallas import tpu_sc as plsc

B, C, H, W = 4, 384, 24, 24
HW = H * W
C2 = 2 * C
NK = 9
NTAP = 4 * NK
NWORK = 32
RPW = B * HW // NWORK
CHUNK = 6
NBUF = 2
NCHUNK = RPW // CHUNK
LANES = 16
NSLICE = C // LANES


def _stage1_body(x_ref, sim_ref, w9_ref, b_ref, idx_ref, wt_ref, xt_ref):
    b = pl.program_id(0)
    xx = x_ref[0]
    tw = jnp.sum(sim_ref[0], axis=0, keepdims=True)
    cc = jnp.concatenate([xx, xx * tw], axis=0)

    pcol = lax.broadcasted_iota(jnp.int32, (1, HW), 1)
    ic = pcol // W
    jc = pcol % W

    acc = jnp.broadcast_to(b_ref[0][:, None], (27, HW)).astype(jnp.float32)
    for t in range(9):
        dy, dx = t // 3 - 1, t % 3 - 1
        s = dy * W + dx
        mt = lax.dot_general(w9_ref[t], cc, (((1,), (0,)), ((), ())),
                             preferred_element_type=jnp.float32)
        if s > 0:
            sh = jnp.concatenate(
                [mt[:, s:], jnp.zeros((27, s), jnp.float32)], axis=1)
        elif s < 0:
            sh = jnp.concatenate(
                [jnp.zeros((27, -s), jnp.float32), mt[:, :s]], axis=1)
        else:
            sh = mt
        mask = ((ic + dy >= 0) & (ic + dy <= H - 1)
                & (jc + dx >= 0) & (jc + dx <= W - 1))
        acc = acc + jnp.where(mask, sh, 0.0)

    off = acc[:18].reshape(NK, 2, HW)
    mod = jax.nn.sigmoid(acc[18:])
    oh = off[:, 0, :] + ic.astype(jnp.float32)
    ow = off[:, 1, :] + jc.astype(jnp.float32)
    valid = ((oh >= 0) & (oh <= H - 1) & (ow >= 0) & (ow <= W - 1))
    h0 = jnp.clip(jnp.floor(oh), 0, H - 1)
    w0 = jnp.clip(jnp.floor(ow), 0, W - 1)
    lh = oh - h0
    lw = ow - w0
    h1 = jnp.minimum(h0 + 1, H - 1)
    w1 = jnp.minimum(w0 + 1, W - 1)

    m = mod * jnp.where(valid, 1.0, 0.0)
    w_taps = jnp.stack(
        [(1 - lh) * (1 - lw) * m, lh * (1 - lw) * m,
         (1 - lh) * lw * m, lh * lw * m], axis=1).reshape(NTAP, HW)
    base = (b * HW).astype(jnp.float32)
    i_taps = jnp.stack(
        [h0 * W + w0, h1 * W + w0, h0 * W + w1, h1 * W + w1],
        axis=1).reshape(NTAP, HW) + base

    idx_ref[0] = jnp.transpose(i_taps, (1, 0)).astype(jnp.int32)
    wt_ref[0] = jnp.transpose(w_taps, (1, 0))
    xtT = jnp.transpose(xx, (1, 0))
    xb = lax.bitcast_convert_type(xtT, jnp.uint32)
    rnd = (xb + jnp.uint32(0x7FFF) + ((xb >> 16) & jnp.uint32(1))) >> 16
    xt_ref[0] = rnd[:, :C // 2] | (rnd[:, C // 2:] << 16)


def _stage1(x2, sim, w9, b27):
    return pl.pallas_call(
        _stage1_body,
        grid=(B,),
        in_specs=[
            pl.BlockSpec((1, C, HW), lambda b: (b, 0, 0)),
            pl.BlockSpec((1, HW, HW), lambda b: (b, 0, 0)),
            pl.BlockSpec((9, 27, C2), lambda b: (0, 0, 0)),
            pl.BlockSpec((1, 27), lambda b: (0, 0)),
        ],
        out_specs=[
            pl.BlockSpec((1, HW, NTAP), lambda b: (b, 0, 0)),
            pl.BlockSpec((1, HW, NTAP), lambda b: (b, 0, 0)),
            pl.BlockSpec((1, HW, C // 2), lambda b: (b, 0, 0)),
        ],
        out_shape=[
            jax.ShapeDtypeStruct((B, HW, NTAP), jnp.int32),
            jax.ShapeDtypeStruct((B, HW, NTAP), jnp.float32),
            jax.ShapeDtypeStruct((B, HW, C // 2), jnp.uint32),
        ],
    )(x2, sim, w9, b27)


def _sc_body(table_hbm, idx_hbm, w_hbm, out_hbm,
             idx_v, w_v, rows_v, acc_v, gsem0, gsem1, gsem2):
    wid = lax.axis_index("s") * 2 + lax.axis_index("c")
    base_r = wid * RPW
    pltpu.sync_copy(idx_hbm.at[pl.ds(base_r * NTAP, RPW * NTAP)], idx_v)
    pltpu.sync_copy(w_hbm.at[pl.ds(base_r * NTAP, RPW * NTAP)], w_v)

    CT = CHUNK * NTAP
    NPAIR = C // 32
    gsems = (gsem0, gsem1, gsem2)

    @pl.loop(0, NCHUNK, step=NBUF)
    def _outer(h):
        handles = [
            pltpu.async_copy(
                table_hbm.at[idx_v.at[pl.ds((h + b) * CT, CT)]],
                rows_v.at[b], gsems[b])
            for b in range(NBUF)
        ]
        for b in range(NBUF):
            handles[b].wait()
            g = h + b
            for rloc in range(CHUNK):
                r = g * CHUNK + rloc

                init = tuple(jnp.zeros((LANES,), jnp.float32)
                             for _ in range(NSLICE))

                @plsc.parallel_loop(0, NTAP // 2, carry=init)
                def acc(i, carr, _b=b, _rloc=rloc):
                    t0 = i * 2
                    wv0 = plsc.load_gather(
                        w_v, [jnp.broadcast_to(r * NTAP + t0, (LANES,))])
                    wv1 = plsc.load_gather(
                        w_v, [jnp.broadcast_to(r * NTAP + t0 + 1, (LANES,))])
                    wb0 = plsc.pack(wv0, wv0,
                                    format=plsc.PackFormat.INTERLEAVED)
                    wb1 = plsc.pack(wv1, wv1,
                                    format=plsc.PackFormat.INTERLEAVED)
                    new = []
                    for s2 in range(NPAIR):
                        u0 = rows_v[_b, _rloc * NTAP + t0,
                                    pl.ds(LANES * s2, LANES)]
                        u1 = rows_v[_b, _rloc * NTAP + t0 + 1,
                                    pl.ds(LANES * s2, LANES)]
                        pr = (wb0 * plsc.bitcast(u0, jnp.bfloat16)
                              + wb1 * plsc.bitcast(u1, jnp.bfloat16))
                        pu = plsc.bitcast(pr, jnp.uint32)
                        ev = plsc.bitcast(pu << 16, jnp.float32)
                        od = plsc.bitcast(pu & jnp.uint32(0xFFFF0000),
                                          jnp.float32)
                        new.append(carr[2 * s2] + ev)
                        new.append(carr[2 * s2 + 1] + od)
                    return tuple(new)
                for s2 in range(NPAIR):
                    acc_v[pl.ds(rloc * C + LANES * s2, LANES)] = acc[2 * s2]
                    acc_v[pl.ds(rloc * C + C // 2 + LANES * s2,
                                LANES)] = acc[2 * s2 + 1]
            pltpu.sync_copy(
                acc_v,
                out_hbm.at[pl.ds((base_r + g * CHUNK) * C, CHUNK * C)])


def _sc_gather_accum(table, idx, wts):
    mesh = plsc.VectorSubcoreMesh(core_axis_name="c", subcore_axis_name="s")
    cp = pltpu.CompilerParams()
    if "needs_layout_passes" in pltpu.CompilerParams.__dataclass_fields__:
        cp = dataclasses.replace(cp, needs_layout_passes=False)
    if "use_tc_tiling_on_sc" in pltpu.CompilerParams.__dataclass_fields__:
        cp = dataclasses.replace(cp, use_tc_tiling_on_sc=False)
    f = pl.kernel(
        _sc_body,
        mesh=mesh,
        out_type=jax.ShapeDtypeStruct((B * HW * C,), jnp.float32),
        scratch_types=[
            pltpu.VMEM((RPW * NTAP,), jnp.int32),
            pltpu.VMEM((RPW * NTAP,), jnp.float32),
            pltpu.VMEM((NBUF, CHUNK * NTAP, C // 2), jnp.uint32),
            pltpu.VMEM((CHUNK * C,), jnp.float32),
            pltpu.SemaphoreType.DMA,
            pltpu.SemaphoreType.DMA,
            pltpu.SemaphoreType.DMA,
        ],
        compiler_params=cp,
    )
    return f(table, idx, wts)


def _tr_body(in_ref, out_ref):
    out_ref[0] = jnp.transpose(in_ref[0], (1, 0))


def _transpose_back(rows):
    return pl.pallas_call(
        _tr_body,
        grid=(B,),
        in_specs=[pl.BlockSpec((1, HW, C), lambda b: (b, 0, 0))],
        out_specs=pl.BlockSpec((1, C, HW), lambda b: (b, 0, 0)),
        out_shape=jax.ShapeDtypeStruct((B, C, HW), jnp.float32),
    )(rows)


def kernel(x, similarity_map, offset_w, offset_b, mod_w, mod_b):
    x2 = x.reshape(B, C, HW)
    w9 = (jnp.concatenate([offset_w, mod_w], axis=0)
          .transpose(2, 3, 0, 1).reshape(9, 27, C2))
    b27 = jnp.concatenate([offset_b, mod_b], axis=0).reshape(1, 27)
    idx, wts, x_t = _stage1(x2, similarity_map, w9, b27)
    out_rows = _sc_gather_accum(
        x_t.reshape(B * HW, C // 2), idx.reshape(-1), wts.reshape(-1))
    out = _transpose_back(out_rows.reshape(B, HW, C))
    return out.reshape(B, C, H, W)


# speedup vs baseline: 1.0029x; 1.0029x over previous
import dataclasses
import functools

import jax
import jax.numpy as jnp
from jax import lax
from jax.experimental import pallas as pl
from jax.experimental.pallas import tpu as pltpu
from jax.experimental.pallas import tpu_sc as plsc

B, C, H, W = 4, 384, 24, 24
HW = H * W
C2 = 2 * C
NK = 9
NTAP = 4 * NK
NWORK = 32
RPW = B * HW // NWORK
CHUNK = 4
NBUF = 3
NCHUNK = RPW // CHUNK
LANES = 16
NSLICE = C // LANES


def _stage1_body(x_ref, sim_ref, w9_ref, b_ref, idx_ref, wt_ref, xt_ref):
    b = pl.program_id(0)
    xx = x_ref[0]
    tw = jnp.sum(sim_ref[0], axis=0, keepdims=True)
    cc = jnp.concatenate([xx, xx * tw], axis=0)

    pcol = lax.broadcasted_iota(jnp.int32, (1, HW), 1)
    ic = pcol // W
    jc = pcol % W

    acc = jnp.broadcast_to(b_ref[0][:, None], (27, HW)).astype(jnp.float32)
    for t in range(9):
        dy, dx = t // 3 - 1, t % 3 - 1
        s = dy * W + dx
        mt = lax.dot_general(w9_ref[t], cc, (((1,), (0,)), ((), ())),
                             preferred_element_type=jnp.float32)
        if s > 0:
            sh = jnp.concatenate(
                [mt[:, s:], jnp.zeros((27, s), jnp.float32)], axis=1)
        elif s < 0:
            sh = jnp.concatenate(
                [jnp.zeros((27, -s), jnp.float32), mt[:, :s]], axis=1)
        else:
            sh = mt
        mask = ((ic + dy >= 0) & (ic + dy <= H - 1)
                & (jc + dx >= 0) & (jc + dx <= W - 1))
        acc = acc + jnp.where(mask, sh, 0.0)

    off = acc[:18].reshape(NK, 2, HW)
    mod = jax.nn.sigmoid(acc[18:])
    oh = off[:, 0, :] + ic.astype(jnp.float32)
    ow = off[:, 1, :] + jc.astype(jnp.float32)
    valid = ((oh >= 0) & (oh <= H - 1) & (ow >= 0) & (ow <= W - 1))
    h0 = jnp.clip(jnp.floor(oh), 0, H - 1)
    w0 = jnp.clip(jnp.floor(ow), 0, W - 1)
    lh = oh - h0
    lw = ow - w0
    h1 = jnp.minimum(h0 + 1, H - 1)
    w1 = jnp.minimum(w0 + 1, W - 1)

    m = mod * jnp.where(valid, 1.0, 0.0)
    w_taps = jnp.stack(
        [(1 - lh) * (1 - lw) * m, lh * (1 - lw) * m,
         (1 - lh) * lw * m, lh * lw * m], axis=1).reshape(NTAP, HW)
    base = (b * HW).astype(jnp.float32)
    i_taps = jnp.stack(
        [h0 * W + w0, h1 * W + w0, h0 * W + w1, h1 * W + w1],
        axis=1).reshape(NTAP, HW) + base

    idx_ref[0] = jnp.transpose(i_taps, (1, 0)).astype(jnp.int32)
    wt_ref[0] = jnp.transpose(w_taps, (1, 0))
    xtT = jnp.transpose(xx, (1, 0))
    xb = lax.bitcast_convert_type(xtT, jnp.uint32)
    rnd = (xb + jnp.uint32(0x7FFF) + ((xb >> 16) & jnp.uint32(1))) >> 16
    xt_ref[0] = rnd[:, :C // 2] | (rnd[:, C // 2:] << 16)


def _stage1(x2, sim, w9, b27):
    return pl.pallas_call(
        _stage1_body,
        grid=(B,),
        in_specs=[
            pl.BlockSpec((1, C, HW), lambda b: (b, 0, 0)),
            pl.BlockSpec((1, HW, HW), lambda b: (b, 0, 0)),
            pl.BlockSpec((9, 27, C2), lambda b: (0, 0, 0)),
            pl.BlockSpec((1, 27), lambda b: (0, 0)),
        ],
        out_specs=[
            pl.BlockSpec((1, HW, NTAP), lambda b: (b, 0, 0)),
            pl.BlockSpec((1, HW, NTAP), lambda b: (b, 0, 0)),
            pl.BlockSpec((1, HW, C // 2), lambda b: (b, 0, 0)),
        ],
        out_shape=[
            jax.ShapeDtypeStruct((B, HW, NTAP), jnp.int32),
            jax.ShapeDtypeStruct((B, HW, NTAP), jnp.float32),
            jax.ShapeDtypeStruct((B, HW, C // 2), jnp.uint32),
        ],
    )(x2, sim, w9, b27)


def _sc_body(table_hbm, idx_hbm, w_hbm, out_hbm,
             idx_v, w_v, rows_v, acc_v, gsem0, gsem1, gsem2):
    wid = lax.axis_index("s") * 2 + lax.axis_index("c")
    base_r = wid * RPW
    pltpu.sync_copy(idx_hbm.at[pl.ds(base_r * NTAP, RPW * NTAP)], idx_v)
    pltpu.sync_copy(w_hbm.at[pl.ds(base_r * NTAP, RPW * NTAP)], w_v)

    CT = CHUNK * NTAP
    NPAIR = C // 32
    gsems = (gsem0, gsem1, gsem2)

    @pl.loop(0, NCHUNK, step=NBUF)
    def _outer(h):
        handles = [
            pltpu.async_copy(
                table_hbm.at[idx_v.at[pl.ds((h + b) * CT, CT)]],
                rows_v.at[b], gsems[b])
            for b in range(NBUF)
        ]
        for b in range(NBUF):
            handles[b].wait()
            g = h + b
            for rloc in range(CHUNK):
                r = g * CHUNK + rloc

                init = tuple(jnp.zeros((LANES,), jnp.float32)
                             for _ in range(NSLICE))

                @plsc.parallel_loop(0, NTAP // 2, carry=init)
                def acc(i, carr, _b=b, _rloc=rloc):
                    t0 = i * 2
                    wv0 = plsc.load_gather(
                        w_v, [jnp.broadcast_to(r * NTAP + t0, (LANES,))])
                    wv1 = plsc.load_gather(
                        w_v, [jnp.broadcast_to(r * NTAP + t0 + 1, (LANES,))])
                    wb0 = plsc.pack(wv0, wv0,
                                    format=plsc.PackFormat.INTERLEAVED)
                    wb1 = plsc.pack(wv1, wv1,
                                    format=plsc.PackFormat.INTERLEAVED)
                    new = []
                    for s2 in range(NPAIR):
                        u0 = rows_v[_b, _rloc * NTAP + t0,
                                    pl.ds(LANES * s2, LANES)]
                        u1 = rows_v[_b, _rloc * NTAP + t0 + 1,
                                    pl.ds(LANES * s2, LANES)]
                        pr = (wb0 * plsc.bitcast(u0, jnp.bfloat16)
                              + wb1 * plsc.bitcast(u1, jnp.bfloat16))
                        pu = plsc.bitcast(pr, jnp.uint32)
                        ev = plsc.bitcast(pu << 16, jnp.float32)
                        od = plsc.bitcast(pu & jnp.uint32(0xFFFF0000),
                                          jnp.float32)
                        new.append(carr[2 * s2] + ev)
                        new.append(carr[2 * s2 + 1] + od)
                    return tuple(new)
                for s2 in range(NPAIR):
                    acc_v[pl.ds(rloc * C + LANES * s2, LANES)] = acc[2 * s2]
                    acc_v[pl.ds(rloc * C + C // 2 + LANES * s2,
                                LANES)] = acc[2 * s2 + 1]
            pltpu.sync_copy(
                acc_v,
                out_hbm.at[pl.ds((base_r + g * CHUNK) * C, CHUNK * C)])


def _sc_gather_accum(table, idx, wts):
    mesh = plsc.VectorSubcoreMesh(core_axis_name="c", subcore_axis_name="s")
    cp = pltpu.CompilerParams()
    if "needs_layout_passes" in pltpu.CompilerParams.__dataclass_fields__:
        cp = dataclasses.replace(cp, needs_layout_passes=False)
    if "use_tc_tiling_on_sc" in pltpu.CompilerParams.__dataclass_fields__:
        cp = dataclasses.replace(cp, use_tc_tiling_on_sc=False)
    f = pl.kernel(
        _sc_body,
        mesh=mesh,
        out_type=jax.ShapeDtypeStruct((B * HW * C,), jnp.float32),
        scratch_types=[
            pltpu.VMEM((RPW * NTAP,), jnp.int32),
            pltpu.VMEM((RPW * NTAP,), jnp.float32),
            pltpu.VMEM((NBUF, CHUNK * NTAP, C // 2), jnp.uint32),
            pltpu.VMEM((CHUNK * C,), jnp.float32),
            pltpu.SemaphoreType.DMA,
            pltpu.SemaphoreType.DMA,
            pltpu.SemaphoreType.DMA,
        ],
        compiler_params=cp,
    )
    return f(table, idx, wts)


def _tr_body(in_ref, out_ref):
    out_ref[0] = jnp.transpose(in_ref[0], (1, 0))


def _transpose_back(rows):
    return pl.pallas_call(
        _tr_body,
        grid=(B,),
        in_specs=[pl.BlockSpec((1, HW, C), lambda b: (b, 0, 0))],
        out_specs=pl.BlockSpec((1, C, HW), lambda b: (b, 0, 0)),
        out_shape=jax.ShapeDtypeStruct((B, C, HW), jnp.float32),
    )(rows)


def kernel(x, similarity_map, offset_w, offset_b, mod_w, mod_b):
    x2 = x.reshape(B, C, HW)
    w9 = (jnp.concatenate([offset_w, mod_w], axis=0)
          .transpose(2, 3, 0, 1).reshape(9, 27, C2))
    b27 = jnp.concatenate([offset_b, mod_b], axis=0).reshape(1, 27)
    idx, wts, x_t = _stage1(x2, similarity_map, w9, b27)
    out_rows = _sc_gather_accum(
        x_t.reshape(B * HW, C // 2), idx.reshape(-1), wts.reshape(-1))
    out = _transpose_back(out_rows.reshape(B, HW, C))
    return out.reshape(B, C, H, W)
